# baseline (speedup 1.0000x reference)
import functools

import jax
import jax.numpy as jnp
from jax.experimental import pallas as pl
from jax.experimental.pallas import tpu as pltpu

_TB = 512


def _round_up(n, m):
    return (n + m - 1) // m * m


def _conv1_weight(w):
    wk = w.astype(jnp.bfloat16).reshape(32, 5, 5)
    lh = jnp.arange(8)
    ol = jnp.arange(4)
    w_in = jnp.arange(32)
    ow2 = jnp.arange(12)
    p = jnp.arange(2)
    di = lh[:, None] - ol[None, :]
    ow = 2 * ow2[None, :] + p[:, None]
    dj = w_in[:, None, None] - ow[None, :, :]
    vi = (di >= 0) & (di < 5)
    vj = (dj >= 0) & (dj < 5)
    t = wk[:, jnp.clip(di, 0, 4), :]
    t = t[:, :, :, jnp.clip(dj, 0, 4)]
    mask = vi[None, :, :, None, None, None] & vj[None, None, None, :, :, :]
    t = jnp.where(mask, t, 0.0)
    t = jnp.transpose(t, (1, 3, 4, 2, 5, 0))
    return t.reshape(256, 3072).astype(jnp.bfloat16)


def _conv2_weight(w):
    wl = jnp.arange(8)
    p = jnp.arange(2)
    ow2l = jnp.arange(2)
    dj = wl[:, None, None] - 2 * ow2l[None, None, :] - p[None, :, None]
    valid = (dj >= 0) & (dj < 5)
    t = w.astype(jnp.bfloat16)[:, :, :, jnp.clip(dj, 0, 4)]
    t = jnp.where(valid[None, None, None, :, :, :], t, 0.0)
    t = jnp.transpose(t, (2, 3, 1, 4, 5, 0))
    return t.reshape(5, 256, 256).astype(jnp.bfloat16)


def _fc1_weight(w):
    f = w.shape[1]
    t = w.astype(jnp.bfloat16).reshape(64, 4, 4, f)
    t = jnp.transpose(t, (1, 2, 0, 3))
    return t.reshape(1024, f).astype(jnp.bfloat16)


def _body(x_ref, w1_ref, b1_ref, w2_ref, b2_ref,
          wf1_ref, bf1_ref, wf2_ref, bf2_ref, o_ref, *, num_classes):
    f32 = jnp.float32
    bf16 = jnp.bfloat16

    xb = x_ref[...]

    w1 = w1_ref[...]
    parts = []
    for g in range(6):
        r = jnp.dot(xb[:, g * 128:g * 128 + 256], w1,
                    preferred_element_type=f32).astype(bf16)
        m = jnp.maximum(r[:, :1536], r[:, 1536:])
        parts.append(jnp.maximum(m[:, 0:384], m[:, 384:768]))
        parts.append(jnp.maximum(m[:, 768:1152], m[:, 1152:1536]))
    a1 = jnp.concatenate(parts, axis=1)
    a1 = jnp.maximum(a1 + b1_ref[...], 0.0)

    w2 = w2_ref[...]
    rows = []
    for oh in range(8):
        halves = []
        for half in range(2):
            acc = jnp.dot(a1[:, oh * 384 + 128 * half:
                              oh * 384 + 128 * half + 256], w2[0],
                          preferred_element_type=f32)
            for hl in range(1, 5):
                b = (oh + hl) * 384 + 128 * half
                acc = acc + jnp.dot(a1[:, b:b + 256], w2[hl],
                                    preferred_element_type=f32)
            halves.append(jnp.maximum(acc[:, :128], acc[:, 128:]))
        rows.append(jnp.concatenate(halves, axis=1).astype(bf16))
    a2 = jnp.concatenate(
        [jnp.maximum(rows[2 * i], rows[2 * i + 1]) for i in range(4)],
        axis=1)
    a2 = jnp.maximum(a2 + b2_ref[...], 0.0)

    h = jnp.dot(a2, wf1_ref[...], preferred_element_type=f32)
    h = jnp.maximum(h + bf1_ref[...], 0.0).astype(bf16)
    logits = jnp.dot(h, wf2_ref[...], preferred_element_type=f32) + bf2_ref[...]
    col = jax.lax.broadcasted_iota(jnp.int32, logits.shape, 1)
    logits = jnp.where(col < num_classes, logits, jnp.float32(-1e30))
    mx = jnp.max(logits, axis=1, keepdims=True)
    s = logits - mx
    lse = jnp.log(jnp.sum(jnp.exp(s), axis=1, keepdims=True))
    o_ref[...] = (s - lse)[:, :num_classes]


def kernel(w_conv1, b_conv1, w_conv2, b_conv2, w_fc1, b_fc1, w_fc2, b_fc2, x):
    num_classes = w_fc2.shape[1]
    B = x.shape[0]
    tb = _TB
    nt = pl.cdiv(B, tb)
    b_pad = nt * tb
    ncp = _round_up(max(num_classes, 128), 128)

    xr = x.astype(jnp.bfloat16).reshape(B, 28, 28)
    xr = jnp.pad(xr, ((0, b_pad - B), (0, 0), (0, 4))).reshape(b_pad, 896)

    w1 = _conv1_weight(w_conv1)
    w2 = _conv2_weight(w_conv2)
    wf1 = _fc1_weight(w_fc1)
    b1t = jnp.tile(b_conv1.astype(jnp.bfloat16), 144).reshape(1, 4608)
    b2t = jnp.tile(b_conv2.astype(jnp.bfloat16), 16).reshape(1, 1024)
    bf1 = b_fc1.astype(jnp.float32).reshape(1, -1)
    wf2 = jnp.pad(w_fc2.astype(jnp.bfloat16),
                  ((0, 0), (0, ncp - num_classes)))
    bf2 = jnp.pad(b_fc2.astype(jnp.float32).reshape(1, -1),
                  ((0, 0), (0, ncp - num_classes)))

    def const(shape):
        return pl.BlockSpec(shape, lambda i: (0,) * len(shape))

    flops = 2 * b_pad * (24 * 24 * 25 * 32 + 8 * 8 * 800 * 64
                         + 1024 * 128 + 128 * ncp)
    w_bytes = 2 * (w1.size + w2.size + wf1.size + wf2.size) \
        + 4 * (b1t.size + b2t.size + bf1.size + bf2.size)
    cost = pl.CostEstimate(flops=flops,
                           transcendentals=b_pad * (ncp + 1),
                           bytes_accessed=xr.size * 2 + b_pad * num_classes * 4
                           + w_bytes)

    out = pl.pallas_call(
        functools.partial(_body, num_classes=num_classes),
        out_shape=jax.ShapeDtypeStruct((b_pad, num_classes), jnp.float32),
        grid=(nt,),
        in_specs=[
            pl.BlockSpec((tb, 896), lambda i: (i, 0)),
            const((256, 3072)), const((1, 4608)),
            const((5, 256, 256)), const((1, 1024)),
            const((1024, 128)), const((1, 128)),
            const((128, ncp)), const((1, ncp)),
        ],
        out_specs=pl.BlockSpec((tb, num_classes), lambda i: (i, 0)),
        compiler_params=pltpu.CompilerParams(
            dimension_semantics=("parallel",),
            vmem_limit_bytes=64 * 1024 * 1024),
        cost_estimate=cost,
    )(xr, w1, b1t, w2, b2t, wf1, bf1, wf2, bf2)
    return out[:B] if b_pad != B else out


# speedup vs baseline: 1.1163x; 1.1163x over previous
import functools

import jax
import jax.numpy as jnp
from jax.experimental import pallas as pl
from jax.experimental.pallas import tpu as pltpu

_TB = 1024


def _round_up(n, m):
    return (n + m - 1) // m * m


def _conv1_weight(w):
    wk = w.astype(jnp.bfloat16).reshape(32, 5, 5)
    lh = jnp.arange(8)
    ol = jnp.arange(4)
    w_in = jnp.arange(32)
    ow2 = jnp.arange(12)
    p = jnp.arange(2)
    di = lh[:, None] - ol[None, :]
    ow = 2 * ow2[None, :] + p[:, None]
    dj = w_in[:, None, None] - ow[None, :, :]
    vi = (di >= 0) & (di < 5)
    vj = (dj >= 0) & (dj < 5)
    t = wk[:, jnp.clip(di, 0, 4), :]
    t = t[:, :, :, jnp.clip(dj, 0, 4)]
    mask = vi[None, :, :, None, None, None] & vj[None, None, None, :, :, :]
    t = jnp.where(mask, t, 0.0)
    t = jnp.transpose(t, (1, 3, 4, 2, 5, 0))
    return t.reshape(256, 3072).astype(jnp.bfloat16)


def _conv2_weight(w):
    wl = jnp.arange(8)
    p = jnp.arange(2)
    ow2l = jnp.arange(2)
    dj = wl[:, None, None] - 2 * ow2l[None, None, :] - p[None, :, None]
    valid = (dj >= 0) & (dj < 5)
    t = w.astype(jnp.bfloat16)[:, :, :, jnp.clip(dj, 0, 4)]
    t = jnp.where(valid[None, None, None, :, :, :], t, 0.0)
    t = jnp.transpose(t, (2, 3, 1, 4, 5, 0))
    return t.reshape(5, 256, 256).astype(jnp.bfloat16)


def _fc1_weight(w):
    f = w.shape[1]
    t = w.astype(jnp.bfloat16).reshape(64, 4, 4, f)
    t = jnp.transpose(t, (1, 2, 0, 3))
    return t.reshape(1024, f).astype(jnp.bfloat16)


def _body(x_ref, w1_ref, b1_ref, w2_ref, b2_ref,
          wf1_ref, bf1_ref, wf2_ref, bf2_ref, o_ref, *, num_classes):
    f32 = jnp.float32
    bf16 = jnp.bfloat16

    xs = x_ref[...].astype(bf16)
    xb = jnp.pad(xs, ((0, 0), (0, 0), (0, 4))).reshape(_TB, 896)

    w1 = w1_ref[...]
    parts = []
    for g in range(6):
        r = jnp.dot(xb[:, g * 128:g * 128 + 256], w1,
                    preferred_element_type=f32).astype(bf16)
        m = jnp.maximum(r[:, :1536], r[:, 1536:])
        parts.append(jnp.maximum(m[:, 0:384], m[:, 384:768]))
        parts.append(jnp.maximum(m[:, 768:1152], m[:, 1152:1536]))
    a1 = jnp.concatenate(parts, axis=1)
    a1 = jnp.maximum(a1 + b1_ref[...], 0.0)

    w2 = w2_ref[...]
    rows = []
    for oh in range(8):
        halves = []
        for half in range(2):
            acc = jnp.dot(a1[:, oh * 384 + 128 * half:
                              oh * 384 + 128 * half + 256], w2[0],
                          preferred_element_type=f32)
            for hl in range(1, 5):
                b = (oh + hl) * 384 + 128 * half
                acc = acc + jnp.dot(a1[:, b:b + 256], w2[hl],
                                    preferred_element_type=f32)
            halves.append(jnp.maximum(acc[:, :128], acc[:, 128:]))
        rows.append(jnp.concatenate(halves, axis=1).astype(bf16))
    a2 = jnp.concatenate(
        [jnp.maximum(rows[2 * i], rows[2 * i + 1]) for i in range(4)],
        axis=1)
    a2 = jnp.maximum(a2 + b2_ref[...], 0.0)

    h = jnp.dot(a2, wf1_ref[...], preferred_element_type=f32)
    h = jnp.maximum(h + bf1_ref[...], 0.0).astype(bf16)
    logits = jnp.dot(h, wf2_ref[...], preferred_element_type=f32) + bf2_ref[...]
    col = jax.lax.broadcasted_iota(jnp.int32, logits.shape, 1)
    logits = jnp.where(col < num_classes, logits, jnp.float32(-1e30))
    mx = jnp.max(logits, axis=1, keepdims=True)
    s = logits - mx
    lse = jnp.log(jnp.sum(jnp.exp(s), axis=1, keepdims=True))
    o_ref[...] = (s - lse)[:, :num_classes]


def kernel(w_conv1, b_conv1, w_conv2, b_conv2, w_fc1, b_fc1, w_fc2, b_fc2, x):
    num_classes = w_fc2.shape[1]
    B = x.shape[0]
    tb = _TB
    nt = pl.cdiv(B, tb)
    b_pad = nt * tb
    ncp = _round_up(max(num_classes, 128), 128)

    xr = x.reshape(B, 28, 28)
    if b_pad != B:
        xr = jnp.pad(xr, ((0, b_pad - B), (0, 0), (0, 0)))

    w1 = _conv1_weight(w_conv1)
    w2 = _conv2_weight(w_conv2)
    wf1 = _fc1_weight(w_fc1)
    b1t = jnp.tile(b_conv1.astype(jnp.bfloat16), 144).reshape(1, 4608)
    b2t = jnp.tile(b_conv2.astype(jnp.bfloat16), 16).reshape(1, 1024)
    bf1 = b_fc1.astype(jnp.float32).reshape(1, -1)
    wf2 = jnp.pad(w_fc2.astype(jnp.bfloat16),
                  ((0, 0), (0, ncp - num_classes)))
    bf2 = jnp.pad(b_fc2.astype(jnp.float32).reshape(1, -1),
                  ((0, 0), (0, ncp - num_classes)))

    def const(shape):
        return pl.BlockSpec(shape, lambda i: (0,) * len(shape))

    flops = 2 * b_pad * (24 * 24 * 25 * 32 + 8 * 8 * 800 * 64
                         + 1024 * 128 + 128 * ncp)
    w_bytes = 2 * (w1.size + w2.size + wf1.size + wf2.size) \
        + 4 * (b1t.size + b2t.size + bf1.size + bf2.size)
    cost = pl.CostEstimate(flops=flops,
                           transcendentals=b_pad * (ncp + 1),
                           bytes_accessed=xr.size * 4 + b_pad * num_classes * 4
                           + w_bytes)

    out = pl.pallas_call(
        functools.partial(_body, num_classes=num_classes),
        out_shape=jax.ShapeDtypeStruct((b_pad, num_classes), jnp.float32),
        grid=(nt,),
        in_specs=[
            pl.BlockSpec((tb, 28, 28), lambda i: (i, 0, 0)),
            const((256, 3072)), const((1, 4608)),
            const((5, 256, 256)), const((1, 1024)),
            const((1024, 128)), const((1, 128)),
            const((128, ncp)), const((1, ncp)),
        ],
        out_specs=pl.BlockSpec((tb, num_classes), lambda i: (i, 0)),
        compiler_params=pltpu.CompilerParams(
            dimension_semantics=("parallel",),
            vmem_limit_bytes=64 * 1024 * 1024),
        cost_estimate=cost,
    )(xr, w1, b1t, w2, b2t, wf1, bf1, wf2, bf2)
    return out[:B] if b_pad != B else out


# speedup vs baseline: 1.1491x; 1.0294x over previous
import functools

import jax
import jax.numpy as jnp
from jax.experimental import pallas as pl
from jax.experimental.pallas import tpu as pltpu

_TB = 1024


def _round_up(n, m):
    return (n + m - 1) // m * m


def _fc1_weight(w):
    f = w.shape[1]
    t = w.astype(jnp.bfloat16).reshape(64, 4, 4, f)
    t = jnp.transpose(t, (1, 2, 0, 3))
    return t.reshape(1024, f).astype(jnp.bfloat16)


def _prep_body(w1_ref, w2_ref, b1_ref, b2_ref,
               W1_ref, W2_ref, B1_ref, B2_ref):
    bf16 = jnp.bfloat16
    w1r = w1_ref[...].astype(bf16)
    S = []
    for di in range(5):
        taps = w1r[di * 5:di * 5 + 5, :]
        chunks = []
        for p in range(2):
            for ow2 in range(12):
                s = 2 * ow2 + p
                chunks.append(jnp.pad(taps, ((s, 27 - s), (0, 0))))
        S.append(jnp.concatenate(chunks, axis=1))
    zero384 = jnp.zeros((32, 384), bf16)
    row_blocks = []
    for lh in range(8):
        lane_chunks = []
        for p in range(2):
            for ol in range(4):
                di = lh - ol
                lane_chunks.append(S[di][:, p * 384:(p + 1) * 384]
                                   if 0 <= di < 5 else zero384)
        row_blocks.append(jnp.concatenate(lane_chunks, axis=1))
    W1_ref[...] = jnp.concatenate(row_blocks, axis=0)

    w2r = w2_ref[...].astype(bf16)
    for hl in range(5):
        v = w2r[hl]
        chunks = []
        for p in range(2):
            for ow2l in range(2):
                s = 32 * (2 * ow2l + p)
                chunks.append(jnp.pad(v, ((s, 96 - s), (0, 0))))
        W2_ref[hl] = jnp.concatenate(chunks, axis=1)

    B1_ref[...] = jnp.tile(b1_ref[...].astype(bf16), (1, 144))
    B2_ref[...] = jnp.tile(b2_ref[...].astype(bf16), (1, 16))


def _body(x_ref, w1_ref, b1_ref, w2_ref, b2_ref,
          wf1_ref, bf1_ref, wf2_ref, bf2_ref, o_ref, *, num_classes):
    f32 = jnp.float32
    bf16 = jnp.bfloat16

    xs = x_ref[...].astype(bf16)
    xb = jnp.pad(xs, ((0, 0), (0, 0), (0, 4))).reshape(_TB, 896)

    w1 = w1_ref[...]
    parts = []
    for g in range(6):
        r = jnp.dot(xb[:, g * 128:g * 128 + 256], w1,
                    preferred_element_type=f32).astype(bf16)
        m = jnp.maximum(r[:, :1536], r[:, 1536:])
        parts.append(jnp.maximum(m[:, 0:384], m[:, 384:768]))
        parts.append(jnp.maximum(m[:, 768:1152], m[:, 1152:1536]))
    a1 = jnp.concatenate(parts, axis=1)
    a1 = jnp.maximum(a1 + b1_ref[...], 0.0)

    w2 = w2_ref[...]
    rows = []
    for oh in range(8):
        halves = []
        for half in range(2):
            acc = jnp.dot(a1[:, oh * 384 + 128 * half:
                              oh * 384 + 128 * half + 256], w2[0],
                          preferred_element_type=f32)
            for hl in range(1, 5):
                b = (oh + hl) * 384 + 128 * half
                acc = acc + jnp.dot(a1[:, b:b + 256], w2[hl],
                                    preferred_element_type=f32)
            halves.append(jnp.maximum(acc[:, :128], acc[:, 128:]))
        rows.append(jnp.concatenate(halves, axis=1).astype(bf16))
    a2 = jnp.concatenate(
        [jnp.maximum(rows[2 * i], rows[2 * i + 1]) for i in range(4)],
        axis=1)
    a2 = jnp.maximum(a2 + b2_ref[...], 0.0)

    h = jnp.dot(a2, wf1_ref[...], preferred_element_type=f32)
    h = jnp.maximum(h + bf1_ref[...], 0.0).astype(bf16)
    logits = jnp.dot(h, wf2_ref[...], preferred_element_type=f32) + bf2_ref[...]
    col = jax.lax.broadcasted_iota(jnp.int32, logits.shape, 1)
    logits = jnp.where(col < num_classes, logits, jnp.float32(-1e30))
    mx = jnp.max(logits, axis=1, keepdims=True)
    s = logits - mx
    lse = jnp.log(jnp.sum(jnp.exp(s), axis=1, keepdims=True))
    o_ref[...] = (s - lse)[:, :num_classes]


def kernel(w_conv1, b_conv1, w_conv2, b_conv2, w_fc1, b_fc1, w_fc2, b_fc2, x):
    num_classes = w_fc2.shape[1]
    B = x.shape[0]
    tb = _TB
    nt = pl.cdiv(B, tb)
    b_pad = nt * tb
    ncp = _round_up(max(num_classes, 128), 128)

    xr = x.reshape(B, 28, 28)
    if b_pad != B:
        xr = jnp.pad(xr, ((0, b_pad - B), (0, 0), (0, 0)))

    w1r = w_conv1.reshape(32, 25).T
    w2r = jnp.transpose(w_conv2, (2, 3, 1, 0)).reshape(5, 160, 64)
    w1, w2, b1t, b2t = pl.pallas_call(
        _prep_body,
        out_shape=(jax.ShapeDtypeStruct((256, 3072), jnp.bfloat16),
                   jax.ShapeDtypeStruct((5, 256, 256), jnp.bfloat16),
                   jax.ShapeDtypeStruct((1, 4608), jnp.bfloat16),
                   jax.ShapeDtypeStruct((1, 1024), jnp.bfloat16)),
    )(w1r, w2r, b_conv1.reshape(1, 32), b_conv2.reshape(1, 64))
    wf1 = _fc1_weight(w_fc1)
    bf1 = b_fc1.astype(jnp.float32).reshape(1, -1)
    wf2 = jnp.pad(w_fc2.astype(jnp.bfloat16),
                  ((0, 0), (0, ncp - num_classes)))
    bf2 = jnp.pad(b_fc2.astype(jnp.float32).reshape(1, -1),
                  ((0, 0), (0, ncp - num_classes)))

    def const(shape):
        return pl.BlockSpec(shape, lambda i: (0,) * len(shape))

    flops = 2 * b_pad * (24 * 24 * 25 * 32 + 8 * 8 * 800 * 64
                         + 1024 * 128 + 128 * ncp)
    w_bytes = 2 * (w1.size + w2.size + wf1.size + wf2.size) \
        + 4 * (b1t.size + b2t.size + bf1.size + bf2.size)
    cost = pl.CostEstimate(flops=flops,
                           transcendentals=b_pad * (ncp + 1),
                           bytes_accessed=xr.size * 4 + b_pad * num_classes * 4
                           + w_bytes)

    out = pl.pallas_call(
        functools.partial(_body, num_classes=num_classes),
        out_shape=jax.ShapeDtypeStruct((b_pad, num_classes), jnp.float32),
        grid=(nt,),
        in_specs=[
            pl.BlockSpec((tb, 28, 28), lambda i: (i, 0, 0)),
            const((256, 3072)), const((1, 4608)),
            const((5, 256, 256)), const((1, 1024)),
            const((1024, 128)), const((1, 128)),
            const((128, ncp)), const((1, ncp)),
        ],
        out_specs=pl.BlockSpec((tb, num_classes), lambda i: (i, 0)),
        compiler_params=pltpu.CompilerParams(
            dimension_semantics=("parallel",),
            vmem_limit_bytes=64 * 1024 * 1024),
        cost_estimate=cost,
    )(xr, w1, b1t, w2, b2t, wf1, bf1, wf2, bf2)
    return out[:B] if b_pad != B else out


# speedup vs baseline: 1.1500x; 1.0007x over previous
import functools

import jax
import jax.numpy as jnp
from jax.experimental import pallas as pl
from jax.experimental.pallas import tpu as pltpu

_TB = 1024


def _round_up(n, m):
    return (n + m - 1) // m * m


def _prep_body(w1_ref, w2_ref, b1_ref, b2_ref, wf1_ref, wf2_ref, bf2_ref,
               W1_ref, W2_ref, B1_ref, B2_ref, WF1_ref, WF2_ref, BF2_ref,
               *, ncp, num_classes):
    bf16 = jnp.bfloat16
    w1r = w1_ref[...].astype(bf16)
    S = []
    for di in range(5):
        taps = w1r[di * 5:di * 5 + 5, :]
        chunks = []
        for p in range(2):
            for ow2 in range(12):
                s = 2 * ow2 + p
                chunks.append(jnp.pad(taps, ((s, 27 - s), (0, 0))))
        S.append(jnp.concatenate(chunks, axis=1))
    zero384 = jnp.zeros((32, 384), bf16)
    row_blocks = []
    for lh in range(8):
        lane_chunks = []
        for p in range(2):
            for ol in range(4):
                di = lh - ol
                lane_chunks.append(S[di][:, p * 384:(p + 1) * 384]
                                   if 0 <= di < 5 else zero384)
        row_blocks.append(jnp.concatenate(lane_chunks, axis=1))
    W1_ref[...] = jnp.concatenate(row_blocks, axis=0)

    w2r = w2_ref[...].astype(bf16)
    for hl in range(5):
        v = w2r[hl]
        chunks = []
        for p in range(2):
            for ow2l in range(2):
                s = 32 * (2 * ow2l + p)
                chunks.append(jnp.pad(v, ((s, 96 - s), (0, 0))))
        W2_ref[hl] = jnp.concatenate(chunks, axis=1)

    B1_ref[...] = jnp.tile(b1_ref[...].astype(bf16), (1, 144))
    B2_ref[...] = jnp.tile(b2_ref[...].astype(bf16), (1, 16))

    wf1 = wf1_ref[...].astype(bf16)
    WF1_ref[...] = jnp.concatenate([wf1[:, j, :] for j in range(16)], axis=0)
    pad_n = ncp - num_classes
    WF2_ref[...] = jnp.concatenate(
        [wf2_ref[...].astype(bf16),
         jnp.zeros((128, pad_n), bf16)], axis=1) if pad_n else \
        wf2_ref[...].astype(bf16)
    BF2_ref[...] = jnp.concatenate(
        [bf2_ref[...], jnp.zeros((1, pad_n), jnp.float32)], axis=1) \
        if pad_n else bf2_ref[...]


def _body(x_ref, w1_ref, b1_ref, w2_ref, b2_ref,
          wf1_ref, bf1_ref, wf2_ref, bf2_ref, o_ref, *, num_classes):
    f32 = jnp.float32
    bf16 = jnp.bfloat16

    xs = x_ref[...].astype(bf16)
    xb = jnp.pad(xs, ((0, 0), (0, 0), (0, 4))).reshape(_TB, 896)

    w1 = w1_ref[...]
    parts = []
    for g in range(6):
        r = jnp.dot(xb[:, g * 128:g * 128 + 256], w1,
                    preferred_element_type=f32).astype(bf16)
        m = jnp.maximum(r[:, :1536], r[:, 1536:])
        parts.append(jnp.maximum(m[:, 0:384], m[:, 384:768]))
        parts.append(jnp.maximum(m[:, 768:1152], m[:, 1152:1536]))
    a1 = jnp.concatenate(parts, axis=1)
    a1 = jnp.maximum(a1 + b1_ref[...], 0.0)

    w2 = w2_ref[...]
    rows = []
    for oh in range(8):
        halves = []
        for half in range(2):
            acc = jnp.dot(a1[:, oh * 384 + 128 * half:
                              oh * 384 + 128 * half + 256], w2[0],
                          preferred_element_type=f32)
            for hl in range(1, 5):
                b = (oh + hl) * 384 + 128 * half
                acc = acc + jnp.dot(a1[:, b:b + 256], w2[hl],
                                    preferred_element_type=f32)
            halves.append(jnp.maximum(acc[:, :128], acc[:, 128:]))
        rows.append(jnp.concatenate(halves, axis=1).astype(bf16))
    a2 = jnp.concatenate(
        [jnp.maximum(rows[2 * i], rows[2 * i + 1]) for i in range(4)],
        axis=1)
    a2 = jnp.maximum(a2 + b2_ref[...], 0.0)

    h = jnp.dot(a2, wf1_ref[...], preferred_element_type=f32)
    h = jnp.maximum(h + bf1_ref[...], 0.0).astype(bf16)
    logits = jnp.dot(h, wf2_ref[...], preferred_element_type=f32) + bf2_ref[...]
    col = jax.lax.broadcasted_iota(jnp.int32, logits.shape, 1)
    logits = jnp.where(col < num_classes, logits, jnp.float32(-1e30))
    mx = jnp.max(logits, axis=1, keepdims=True)
    s = logits - mx
    lse = jnp.log(jnp.sum(jnp.exp(s), axis=1, keepdims=True))
    o_ref[...] = (s - lse)[:, :num_classes]


def kernel(w_conv1, b_conv1, w_conv2, b_conv2, w_fc1, b_fc1, w_fc2, b_fc2, x):
    num_classes = w_fc2.shape[1]
    B = x.shape[0]
    tb = _TB
    nt = pl.cdiv(B, tb)
    b_pad = nt * tb
    ncp = _round_up(max(num_classes, 128), 128)

    xr = x.reshape(B, 28, 28)
    if b_pad != B:
        xr = jnp.pad(xr, ((0, b_pad - B), (0, 0), (0, 0)))

    w1r = w_conv1.reshape(32, 25).T
    w2r = jnp.transpose(w_conv2, (2, 3, 1, 0)).reshape(5, 160, 64)
    w1, w2, b1t, b2t, wf1, wf2, bf2 = pl.pallas_call(
        functools.partial(_prep_body, ncp=ncp, num_classes=num_classes),
        out_shape=(jax.ShapeDtypeStruct((256, 3072), jnp.bfloat16),
                   jax.ShapeDtypeStruct((5, 256, 256), jnp.bfloat16),
                   jax.ShapeDtypeStruct((1, 4608), jnp.bfloat16),
                   jax.ShapeDtypeStruct((1, 1024), jnp.bfloat16),
                   jax.ShapeDtypeStruct((1024, 128), jnp.bfloat16),
                   jax.ShapeDtypeStruct((128, ncp), jnp.bfloat16),
                   jax.ShapeDtypeStruct((1, ncp), jnp.float32)),
    )(w1r, w2r, b_conv1.reshape(1, 32), b_conv2.reshape(1, 64),
      w_fc1.reshape(64, 16, 128), w_fc2.astype(jnp.float32),
      b_fc2.astype(jnp.float32).reshape(1, -1))
    bf1 = b_fc1.astype(jnp.float32).reshape(1, -1)

    def const(shape):
        return pl.BlockSpec(shape, lambda i: (0,) * len(shape))

    flops = 2 * b_pad * (24 * 24 * 25 * 32 + 8 * 8 * 800 * 64
                         + 1024 * 128 + 128 * ncp)
    w_bytes = 2 * (w1.size + w2.size + wf1.size + wf2.size) \
        + 4 * (b1t.size + b2t.size + bf1.size + bf2.size)
    cost = pl.CostEstimate(flops=flops,
                           transcendentals=b_pad * (ncp + 1),
                           bytes_accessed=xr.size * 4 + b_pad * num_classes * 4
                           + w_bytes)

    out = pl.pallas_call(
        functools.partial(_body, num_classes=num_classes),
        out_shape=jax.ShapeDtypeStruct((b_pad, num_classes), jnp.float32),
        grid=(nt,),
        in_specs=[
            pl.BlockSpec((tb, 28, 28), lambda i: (i, 0, 0)),
            const((256, 3072)), const((1, 4608)),
            const((5, 256, 256)), const((1, 1024)),
            const((1024, 128)), const((1, 128)),
            const((128, ncp)), const((1, ncp)),
        ],
        out_specs=pl.BlockSpec((tb, num_classes), lambda i: (i, 0)),
        compiler_params=pltpu.CompilerParams(
            dimension_semantics=("parallel",),
            vmem_limit_bytes=64 * 1024 * 1024),
        cost_estimate=cost,
    )(xr, w1, b1t, w2, b2t, wf1, bf1, wf2, bf2)
    return out[:B] if b_pad != B else out
